# baseline (speedup 1.0000x reference)
import functools

import jax
import jax.numpy as jnp
from jax.experimental import pallas as pl
from jax.experimental.pallas import tpu as pltpu

_BB = 256
_HB = 128
_CB = 64

_LOG2E = 1.4426950408889634


def _half_pipeline(p2_ref, c_ref, tp_ref, tc_ref, out_ref, ex_ref, r0, n, d):
    rs = slice(r0, r0 + _HB)
    p2 = p2_ref[rs, :]
    cv = c_ref[rs, :] * _LOG2E
    crep = pltpu.repeat(cv, n, axis=1)
    prod = p2 * crep
    sel = jnp.repeat(jnp.eye(n, dtype=jnp.float32), d, axis=0)
    dw = jax.lax.dot(prod, sel)
    tw = _LOG2E / jnp.abs(tp_ref[rs, :] - tc_ref[rs, :])
    w = dw + tw
    e = jnp.exp2(w - jnp.max(w, axis=-1, keepdims=True))
    z = jnp.sum(e, axis=-1, keepdims=True)
    en = e / z
    ent = en.T
    ex_ref[...] = jnp.broadcast_to(ent[:, :, None], (n, _HB, d))
    for k in range(_HB // _CB):
        rows = slice(k * _CB, (k + 1) * _CB)
        orows = slice(r0 + k * _CB, r0 + (k + 1) * _CB)
        acc = ex_ref[0, rows, :] * p2_ref[orows, 0:d]
        for j in range(1, n):
            acc = acc + ex_ref[j, rows, :] * p2_ref[orows, j * d : (j + 1) * d]
        out_ref[orows, :] = acc


def _agg_kernel(p2_ref, c_ref, tp_ref, tc_ref, out_ref, ex0_ref, ex1_ref):
    bb, nd = p2_ref.shape
    d = c_ref.shape[1]
    n = nd // d
    _half_pipeline(p2_ref, c_ref, tp_ref, tc_ref, out_ref, ex0_ref, 0, n, d)
    _half_pipeline(p2_ref, c_ref, tp_ref, tc_ref, out_ref, ex1_ref, _HB, n, d)


@jax.jit
def kernel(pref, c, t_pref, t_c):
    B, N, D = pref.shape
    grid = (B // _BB,)
    out = pl.pallas_call(
        _agg_kernel,
        grid=grid,
        in_specs=[
            pl.BlockSpec((_BB, N * D), lambda i: (i, 0)),
            pl.BlockSpec((_BB, D), lambda i: (i, 0)),
            pl.BlockSpec((_BB, N), lambda i: (i, 0)),
            pl.BlockSpec((_BB, 1), lambda i: (i, 0)),
        ],
        out_specs=pl.BlockSpec((_BB, D), lambda i: (i, 0)),
        out_shape=jax.ShapeDtypeStruct((B, D), pref.dtype),
        scratch_shapes=[
            pltpu.VMEM((N, _HB, D), jnp.float32),
            pltpu.VMEM((N, _HB, D), jnp.float32),
        ],
        compiler_params=pltpu.CompilerParams(
            dimension_semantics=("arbitrary",),
            vmem_limit_bytes=56 * 1024 * 1024,
        ),
        name="softmax_pool_agg",
    )(pref.reshape(B, N * D), c[:, 0, :], t_pref[:, 0, :], t_c)
    return out[:, None, :]


# speedup vs baseline: 2.9971x; 2.9971x over previous
import functools

import jax
import jax.numpy as jnp
from jax.experimental import pallas as pl
from jax.experimental.pallas import tpu as pltpu

_BB = 512

_LOG2E = 1.4426950408889634


def _agg_kernel(pref_ref, c_ref, tp_ref, tc_ref, out_ref):
    bb, n, d = pref_ref.shape
    p = pref_ref[...]
    cv = c_ref[...] * _LOG2E
    prod = p * cv[:, None, :]
    ones = jnp.ones((d, d), dtype=jnp.float32)
    dw = jax.lax.dot(prod.reshape(bb * n, d), ones).reshape(bb, n, d)
    tw = _LOG2E / jnp.abs(tp_ref[...] - tc_ref[...])
    w = dw + tw[:, :, None]
    e = jnp.exp2(w - jnp.max(w, axis=1, keepdims=True))
    num = jnp.sum(e * p, axis=1)
    z = jnp.sum(e, axis=1)
    out_ref[...] = num / z


@jax.jit
def kernel(pref, c, t_pref, t_c):
    B, N, D = pref.shape
    grid = (B // _BB,)
    out = pl.pallas_call(
        _agg_kernel,
        grid=grid,
        in_specs=[
            pl.BlockSpec((_BB, N, D), lambda i: (i, 0, 0)),
            pl.BlockSpec((_BB, D), lambda i: (i, 0)),
            pl.BlockSpec((_BB, N), lambda i: (i, 0)),
            pl.BlockSpec((_BB, 1), lambda i: (i, 0)),
        ],
        out_specs=pl.BlockSpec((_BB, D), lambda i: (i, 0)),
        out_shape=jax.ShapeDtypeStruct((B, D), pref.dtype),
        compiler_params=pltpu.CompilerParams(
            dimension_semantics=("arbitrary",),
            vmem_limit_bytes=56 * 1024 * 1024,
        ),
        name="softmax_pool_agg",
    )(pref, c[:, 0, :], t_pref[:, 0, :], t_c)
    return out[:, None, :]
